# baseline (speedup 1.0000x reference)
import functools

import jax
import jax.numpy as jnp
from jax import lax
from jax.experimental import pallas as pl
from jax.experimental.pallas import tpu as pltpu
from jax.experimental.pallas import tpu_sc as plsc

NUM_ROWS = 1_000_000
DIM = 32
NB = 16384
NS = 50

_TC_PER_W = 4
_SC_PER_CHUNK = 10
_CHUNKS = NS // _SC_PER_CHUNK
_CROWS = 128 * _SC_PER_CHUNK

_mesh = plsc.VectorSubcoreMesh(core_axis_name="c", subcore_axis_name="s")


@functools.partial(
    pl.kernel,
    mesh=_mesh,
    out_type=jax.ShapeDtypeStruct((NS, DIM // 8, NB // 128, 1024),
                                  jnp.float32),
    scratch_types=[
        pltpu.VMEM((6400,), jnp.int32),
        pltpu.VMEM((6400,), jnp.int32),
        pltpu.VMEM((2, _CROWS, DIM), jnp.float32),
        pltpu.VMEM((2, 4096), jnp.float32),
        pltpu.SemaphoreType.DMA,
        [pltpu.SemaphoreType.DMA] * 2,
        [pltpu.SemaphoreType.DMA] * 2,
    ],
    compiler_params=pltpu.CompilerParams(use_tc_tiling_on_sc=False,
                                         needs_layout_passes=False),
)
def _gather_kernel(idx_hbm, table_hbm, out_hbm, raw_v, sm_v, rows_v,
                   panel_v, si, sg, so):
    wid = lax.axis_index("s") * 2 + lax.axis_index("c")

    iota = lax.broadcasted_iota(jnp.int32, (16,), 0)

    def idx_dma(tc_abs):
        return pltpu.async_copy(
            idx_hbm.at[pl.ds(tc_abs * 6400, 6400)], raw_v, si)

    def regroup():
        @plsc.parallel_loop(0, NS, 1, unroll=2)
        def body(s):
            splat = jnp.broadcast_to(s, (16,)).astype(jnp.int32)
            for m in range(8):
                pos = (iota + 16 * m) * NS + splat
                sm_v[pl.ds(s * 128 + 16 * m, 16)] = plsc.load_gather(
                    raw_v, [pos])

    def start_gather(c):
        return pltpu.async_copy(
            table_hbm.at[sm_v.at[pl.ds(c * _CROWS, _CROWS)]],
            rows_v.at[c & 1], sg[c & 1])

    def wait_out(sp):
        for tr in range(4):
            pltpu.make_async_copy(
                panel_v.at[sp, pl.ds(1024 * tr, 1024)],
                out_hbm.at[0, 0, 0], so[sp]).wait()

    def start_out(sp, s, tc_abs):
        for tr in range(4):
            pltpu.async_copy(
                panel_v.at[sp, pl.ds(1024 * tr, 1024)],
                out_hbm.at[s, tr, tc_abs], so[sp])

    def transpose_one(cp, q, sp):
        @plsc.parallel_loop(0, 128, 1, unroll=16)
        def body(bl):
            blsplat = jnp.broadcast_to(bl, (16,)).astype(jnp.int32)
            for h in range(2):
                val = rows_v[cp, q * 128 + bl, pl.ds(16 * h, 16)]
                pos = (16 * h + iota) * 128 + blsplat
                plsc.store_scatter(panel_v.at[sp], [pos], val)

    first_tc = wid * _TC_PER_W
    idx_dma(first_tc).wait()
    for tc in range(_TC_PER_W):
        tc_abs = first_tc + tc
        regroup()
        if tc + 1 < _TC_PER_W:
            d = idx_dma(tc_abs + 1)
        g = [None] * _CHUNKS
        g[0] = start_gather(0)
        for c in range(_CHUNKS):
            g[c].wait()
            if c + 1 < _CHUNKS:
                g[c + 1] = start_gather(c + 1)

            def q_body(jq, carry, _c=c, _tc=tc, _tc_abs=tc_abs):
                for sp in range(2):
                    q = 2 * jq + sp
                    s = _c * _SC_PER_CHUNK + q
                    if _tc == 0 and _c == 0:
                        @pl.when(jq > 0)
                        def _():
                            wait_out(sp)
                    else:
                        wait_out(sp)
                    transpose_one(_c & 1, q, sp)
                    start_out(sp, s, _tc_abs)
                return carry

            lax.fori_loop(0, _SC_PER_CHUNK // 2, q_body, 0)
        if tc + 1 < _TC_PER_W:
            d.wait()

    wait_out(0)
    wait_out(1)


def kernel(inputs, W):
    idx = inputs.reshape(-1).astype(jnp.int32)
    out4 = _gather_kernel(idx, W).reshape(NS, DIM // 8, NB // 128, 8, 128)
    return out4.transpose(2, 4, 0, 1, 3).reshape(NB, NS, DIM)


# speedup vs baseline: 1.0021x; 1.0021x over previous
import functools

import jax
import jax.numpy as jnp
from jax import lax
from jax.experimental import pallas as pl
from jax.experimental.pallas import tpu as pltpu
from jax.experimental.pallas import tpu_sc as plsc

NUM_ROWS = 1_000_000
DIM = 32
NB = 16384
NS = 50

_TC_PER_W = 4
_SC_PER_CHUNK = 10
_CHUNKS = NS // _SC_PER_CHUNK
_CROWS = 128 * _SC_PER_CHUNK

_mesh = plsc.VectorSubcoreMesh(core_axis_name="c", subcore_axis_name="s")


@functools.partial(
    pl.kernel,
    mesh=_mesh,
    out_type=jax.ShapeDtypeStruct((NS, DIM // 8, NB // 128, 1024),
                                  jnp.float32),
    scratch_types=[
        pltpu.VMEM((6400,), jnp.int32),
        pltpu.VMEM((6400,), jnp.int32),
        pltpu.VMEM((2, _CROWS, DIM), jnp.float32),
        pltpu.VMEM((2, 4096), jnp.float32),
        pltpu.SemaphoreType.DMA,
        [pltpu.SemaphoreType.DMA] * 2,
        [pltpu.SemaphoreType.DMA] * 2,
    ],
    compiler_params=pltpu.CompilerParams(use_tc_tiling_on_sc=False,
                                         needs_layout_passes=False,
                                         disable_bounds_checks=True),
)
def _gather_kernel(idx_hbm, table_hbm, out_hbm, raw_v, sm_v, rows_v,
                   panel_v, si, sg, so):
    wid = lax.axis_index("s") * 2 + lax.axis_index("c")

    iota = lax.broadcasted_iota(jnp.int32, (16,), 0)

    def idx_dma(tc_abs):
        return pltpu.async_copy(
            idx_hbm.at[pl.ds(tc_abs * 6400, 6400)], raw_v, si)

    def regroup():
        @plsc.parallel_loop(0, NS, 1, unroll=2)
        def body(s):
            splat = jnp.broadcast_to(s, (16,)).astype(jnp.int32)
            for m in range(8):
                pos = (iota + 16 * m) * NS + splat
                sm_v[pl.ds(s * 128 + 16 * m, 16)] = plsc.load_gather(
                    raw_v, [pos])

    def start_gather(c):
        return pltpu.async_copy(
            table_hbm.at[sm_v.at[pl.ds(c * _CROWS, _CROWS)]],
            rows_v.at[c & 1], sg[c & 1])

    def wait_out(sp):
        for tr in range(4):
            pltpu.make_async_copy(
                panel_v.at[sp, pl.ds(1024 * tr, 1024)],
                out_hbm.at[0, 0, 0], so[sp]).wait()

    def start_out(sp, s, tc_abs):
        for tr in range(4):
            pltpu.async_copy(
                panel_v.at[sp, pl.ds(1024 * tr, 1024)],
                out_hbm.at[s, tr, tc_abs], so[sp])

    def transpose_one(cp, q, sp):
        @plsc.parallel_loop(0, 128, 1, unroll=8)
        def body(bl):
            blsplat = jnp.broadcast_to(bl, (16,)).astype(jnp.int32)
            for h in range(2):
                val = rows_v[cp, q * 128 + bl, pl.ds(16 * h, 16)]
                pos = (16 * h + iota) * 128 + blsplat
                plsc.store_scatter(panel_v.at[sp], [pos], val)

    first_tc = wid * _TC_PER_W
    idx_dma(first_tc).wait()
    for tc in range(_TC_PER_W):
        tc_abs = first_tc + tc
        regroup()
        if tc + 1 < _TC_PER_W:
            d = idx_dma(tc_abs + 1)
        g = [None] * _CHUNKS
        g[0] = start_gather(0)
        for c in range(_CHUNKS):
            g[c].wait()
            if c + 1 < _CHUNKS:
                g[c + 1] = start_gather(c + 1)

            def q_body(jq, carry, _c=c, _tc=tc, _tc_abs=tc_abs):
                for sp in range(2):
                    q = 2 * jq + sp
                    s = _c * _SC_PER_CHUNK + q
                    if _tc == 0 and _c == 0:
                        @pl.when(jq > 0)
                        def _():
                            wait_out(sp)
                    else:
                        wait_out(sp)
                    transpose_one(_c & 1, q, sp)
                    start_out(sp, s, _tc_abs)
                return carry

            lax.fori_loop(0, _SC_PER_CHUNK // 2, q_body, 0)
        if tc + 1 < _TC_PER_W:
            d.wait()

    wait_out(0)
    wait_out(1)


def kernel(inputs, W):
    idx = inputs.reshape(-1).astype(jnp.int32)
    out4 = _gather_kernel(idx, W).reshape(NS, DIM // 8, NB // 128, 8, 128)
    return out4.transpose(2, 4, 0, 1, 3).reshape(NB, NS, DIM)
